# baseline (speedup 1.0000x reference)
import functools

import jax
import jax.numpy as jnp
from jax import lax
from jax.experimental import pallas as pl
from jax.experimental.pallas import tpu as pltpu
from jax.experimental.pallas import tpu_sc as plsc

N = 10000
E = 640000
EB = 128
KJ = 5
NC = 2
NS = 16
NW = NC * NS
RW = 157
ROWS_P = NW * RW
E_PAD = ROWS_P * EB
NIT = 31
TAIL = RW - NIT * KJ
D_PAD = 48
DEG_W = 16
N_T = 10016
ROWS_T = N_T // NS
BN = 400


def _sc_mesh():
    return plsc.VectorSubcoreMesh(core_axis_name="c", subcore_axis_name="s")


_SC_PARAMS = pltpu.CompilerParams(use_tc_tiling_on_sc=False)


def _fill_zeros(ref, rows, width):
    zv = jnp.zeros((16,), jnp.float32)

    def body(r, carry):
        for j in range(width // 16):
            ref[r, pl.ds(j * 16, 16)] = zv
        return carry
    lax.fori_loop(0, rows, body, 0)


def _deg_partials(d1, d2):
    @functools.partial(
        pl.kernel,
        mesh=_sc_mesh(),
        compiler_params=_SC_PARAMS,
        out_type=[jax.ShapeDtypeStruct((NC, N_T, DEG_W), jnp.float32),
                  jax.ShapeDtypeStruct((NC, N_T, DEG_W), jnp.float32)],
        scratch_types=[
            pltpu.VMEM((RW, EB), jnp.int32),
            pltpu.VMEM((EB, DEG_W), jnp.float32),
            pltpu.VMEM((ROWS_T, DEG_W), jnp.float32),
            pltpu.VMEM_SHARED((N_T, DEG_W), jnp.float32),
            pltpu.VMEM_SHARED((N_T, DEG_W), jnp.float32),
        ],
    )
    def kern(d1h, d2h, o1, o2, idxv, onesv, zv, t1, t2):
        cid = lax.axis_index("c")
        sid = lax.axis_index("s")
        wid = sid * NC + cid
        r0 = sid * ROWS_T
        _fill_zeros(zv, ROWS_T, DEG_W)
        one = jnp.ones((16,), jnp.float32)

        def fill_ones(r, carry):
            onesv[r, pl.ds(0, 16)] = one
            return carry
        lax.fori_loop(0, EB, fill_ones, 0)
        pltpu.sync_copy(zv, t1.at[pl.ds(r0, ROWS_T)])
        pltpu.sync_copy(zv, t2.at[pl.ds(r0, ROWS_T)])
        plsc.subcore_barrier()
        for dh, tbl in ((d1h, t1), (d2h, t2)):
            pltpu.sync_copy(dh.at[pl.ds(wid * RW, RW)], idxv)

            def body(it, carry, tbl=tbl):
                for j in range(KJ):
                    pltpu.sync_copy(onesv, tbl.at[idxv.at[it * KJ + j]],
                                    add=True)
                return carry
            lax.fori_loop(0, NIT, body, 0)
            for j in range(TAIL):
                pltpu.sync_copy(onesv, tbl.at[idxv.at[NIT * KJ + j]],
                                add=True)
        plsc.subcore_barrier()
        pltpu.sync_copy(t1.at[pl.ds(r0, ROWS_T)], o1.at[cid, pl.ds(r0, ROWS_T)])
        pltpu.sync_copy(t2.at[pl.ds(r0, ROWS_T)], o2.at[cid, pl.ds(r0, ROWS_T)])

    return kern(d1, d2)


def _edge_agg_one(sh_in, dh_in, xs):
    @functools.partial(
        pl.kernel,
        mesh=_sc_mesh(),
        compiler_params=_SC_PARAMS,
        out_type=jax.ShapeDtypeStruct((NC, N_T, D_PAD), jnp.float32),
        scratch_types=[
            pltpu.VMEM((RW, EB), jnp.int32),
            pltpu.VMEM((RW, EB), jnp.int32),
            pltpu.VMEM((KJ, EB, D_PAD), jnp.float32),
            pltpu.VMEM((ROWS_T, D_PAD), jnp.float32),
            pltpu.VMEM_SHARED((N_T, D_PAD), jnp.float32),
            pltpu.SemaphoreType.DMA,
        ],
    )
    def kern(sh, dh, xh, o1, sv, dv, rowsv, zv, a1, sem):
        cid = lax.axis_index("c")
        sid = lax.axis_index("s")
        wid = sid * NC + cid
        r0 = sid * ROWS_T
        _fill_zeros(zv, ROWS_T, D_PAD)
        pltpu.sync_copy(zv, a1.at[pl.ds(r0, ROWS_T)])
        pltpu.sync_copy(sh.at[pl.ds(wid * RW, RW)], sv)
        pltpu.sync_copy(dh.at[pl.ds(wid * RW, RW)], dv)
        plsc.subcore_barrier()

        def body(it, carry):
            cps = [pltpu.async_copy(xh.at[sv.at[it * KJ + j]],
                                    rowsv.at[j], sem)
                   for j in range(KJ)]
            for c in cps:
                c.wait()
            for j in range(KJ):
                pltpu.sync_copy(rowsv.at[j], a1.at[dv.at[it * KJ + j]],
                                add=True)
            return carry

        lax.fori_loop(0, NIT, body, 0)
        cps = [pltpu.async_copy(xh.at[sv.at[NIT * KJ + j]],
                                rowsv.at[j], sem)
               for j in range(TAIL)]
        for c in cps:
            c.wait()
        for j in range(TAIL):
            pltpu.sync_copy(rowsv.at[j], a1.at[dv.at[NIT * KJ + j]],
                            add=True)
        plsc.subcore_barrier()
        pltpu.sync_copy(a1.at[pl.ds(r0, ROWS_T)], o1.at[cid, pl.ds(r0, ROWS_T)])

    return kern(sh_in, dh_in, xs)


def _enc_body(xc_ref, y_ref, d1_ref, d2_ref, wd_ref, bd_ref, fcw_ref, fcb_ref,
              l1w_ref, l1b_ref, wih_ref, whh_ref, bih_ref, bhh_ref,
              g1w_ref, g2w_ref, xs1_ref, xs2_ref, di1_ref, di2_ref):
    xc = xc_ref[...].reshape(5 * BN, 392)
    co = jnp.maximum(
        jnp.dot(xc, wd_ref[...], preferred_element_type=jnp.float32)
        + bd_ref[...], 0.0)
    f = jnp.dot(co, fcw_ref[...], preferred_element_type=jnp.float32) + fcb_ref[...]
    l = jnp.maximum(
        jnp.dot(f, l1w_ref[...], preferred_element_type=jnp.float32)
        + l1b_ref[...], 0.0)
    wih = wih_ref[...]
    whh = whh_ref[...]
    bih = bih_ref[...]
    bhh = bhh_ref[...]
    h = jnp.zeros((BN, 64), jnp.float32)
    for t in range(4):
        xt = jnp.concatenate([l[t * BN:(t + 1) * BN], y_ref[t]], axis=1)
        gi = jnp.dot(xt, wih, preferred_element_type=jnp.float32) + bih
        gh = jnp.dot(h, whh, preferred_element_type=jnp.float32) + bhh
        r = jax.nn.sigmoid(gi[:, 0:64] + gh[:, 0:64])
        z = jax.nn.sigmoid(gi[:, 64:128] + gh[:, 64:128])
        n = jnp.tanh(gi[:, 128:192] + r * gh[:, 128:192])
        h = (1.0 - z) * n + z * h
    hh = jnp.concatenate([l[4 * BN:5 * BN], h], axis=1)
    pad = jnp.zeros((BN, D_PAD - 40), jnp.float32)
    for d_ref, gw_ref, xs_ref, di_ref in (
            (d1_ref, g1w_ref, xs1_ref, di1_ref),
            (d2_ref, g2w_ref, xs2_ref, di2_ref)):
        deg = d_ref[0, :, 0:1] + d_ref[1, :, 0:1] + 1.0
        dinv = lax.rsqrt(deg)
        xw = jnp.dot(hh, gw_ref[...], preferred_element_type=jnp.float32)
        xs_ref[...] = jnp.concatenate([xw * dinv, pad], axis=1)
        di_ref[...] = dinv


def _encoder(xcT, yT, dp1, dp2, wd, bd, fcp, fcb, l1w, l1b,
             wihT, whhT, bih, bhh, g1w, g2w):
    full = lambda shape: pl.BlockSpec(shape, lambda i: tuple(0 for _ in shape))
    return pl.pallas_call(
        _enc_body,
        grid=(N // BN,),
        in_specs=[
            pl.BlockSpec((5, BN, 392), lambda i: (0, i, 0)),
            pl.BlockSpec((5, BN, 1), lambda i: (0, i, 0)),
            pl.BlockSpec((NC, BN, DEG_W), lambda i: (0, i, 0)),
            pl.BlockSpec((NC, BN, DEG_W), lambda i: (0, i, 0)),
            full((392, 784)),
            full((1, 784)),
            full((784, 80)),
            full((1, 80)),
            full((80, 40)),
            full((1, 40)),
            full((41, 192)),
            full((64, 192)),
            full((1, 192)),
            full((1, 192)),
            full((104, 40)),
            full((104, 40)),
        ],
        out_specs=[
            pl.BlockSpec((BN, D_PAD), lambda i: (i, 0)),
            pl.BlockSpec((BN, D_PAD), lambda i: (i, 0)),
            pl.BlockSpec((BN, 1), lambda i: (i, 0)),
            pl.BlockSpec((BN, 1), lambda i: (i, 0)),
        ],
        out_shape=[
            jax.ShapeDtypeStruct((N, D_PAD), jnp.float32),
            jax.ShapeDtypeStruct((N, D_PAD), jnp.float32),
            jax.ShapeDtypeStruct((N, 1), jnp.float32),
            jax.ShapeDtypeStruct((N, 1), jnp.float32),
        ],
    )(xcT, yT, dp1, dp2, wd, bd, fcp, fcb, l1w, l1b,
      wihT, whhT, bih, bhh, g1w, g2w)


def _comb_body(a1_ref, a2_ref, xs1_ref, xs2_ref, di1_ref, di2_ref,
               b1_ref, b2_ref, mw1_ref, mw2_ref, mb_ref, out_ref):
    g1 = jnp.maximum(
        (a1_ref[0] + a1_ref[1] + xs1_ref[...])[:, :40] * di1_ref[...]
        + b1_ref[...], 0.0)
    g2 = jnp.maximum(
        (a2_ref[0] + a2_ref[1] + xs2_ref[...])[:, :40] * di2_ref[...]
        + b2_ref[...], 0.0)
    out_ref[...] = (
        jnp.dot(g1, mw1_ref[...], preferred_element_type=jnp.float32)
        + jnp.dot(g2, mw2_ref[...], preferred_element_type=jnp.float32)
        + mb_ref[...])


def _combine(a1, a2, xs1, xs2, di1, di2, b1, b2, mw1, mw2, mb):
    return pl.pallas_call(
        _comb_body,
        out_shape=jax.ShapeDtypeStruct((N, 1), jnp.float32),
    )(a1, a2, xs1, xs2, di1, di2, b1, b2, mw1, mw2, mb)


def kernel(x, edge_index, feat_edge_index, conv_w, conv_b, fc_w, fc_b,
           lin1_w, lin1_b, gru_w_ih, gru_w_hh, gru_b_ih, gru_b_hh,
           gcn1_w, gcn1_b, gcn2_w, gcn2_b, mlp_w, mlp_b):
    f32 = jnp.float32
    xcT = x[:, :, 3:].transpose(1, 0, 2)
    yT = x[:, :, 2].T[:, :, None]
    wd = jnp.einsum("pq,ck->pkqc", jnp.eye(49, dtype=f32),
                    conv_w[:, 0, :]).reshape(392, 784)
    bd = jnp.tile(conv_b, 49)[None, :]
    fcp = fc_w.reshape(16, 49, 80).transpose(1, 0, 2).reshape(784, 80)

    pad_src = jnp.zeros((E_PAD - E,), jnp.int32)
    pad_dst = jnp.full((E_PAD - E,), N, jnp.int32)
    s1 = jnp.concatenate([edge_index[0], pad_src]).reshape(ROWS_P, EB)
    d1 = jnp.concatenate([edge_index[1], pad_dst]).reshape(ROWS_P, EB)
    s2 = jnp.concatenate([feat_edge_index[0], pad_src]).reshape(ROWS_P, EB)
    d2 = jnp.concatenate([feat_edge_index[1], pad_dst]).reshape(ROWS_P, EB)

    dp1, dp2 = _deg_partials(d1, d2)
    xs1, xs2, di1, di2 = _encoder(
        xcT, yT, dp1[:, :N], dp2[:, :N], wd, bd, fcp, fc_b[None],
        lin1_w, lin1_b[None], gru_w_ih.T, gru_w_hh.T, gru_b_ih[None],
        gru_b_hh[None], gcn1_w, gcn2_w)
    a1 = _edge_agg_one(s1, d1, xs1)
    a2 = _edge_agg_one(s2, d2, xs2)
    return _combine(a1[:, :N], a2[:, :N], xs1, xs2, di1, di2,
                    gcn1_b[None], gcn2_b[None],
                    mlp_w[:40], mlp_w[40:], mlp_b[None])


# speedup vs baseline: 1.2946x; 1.2946x over previous
import functools

import jax
import jax.numpy as jnp
from jax import lax
from jax.experimental import pallas as pl
from jax.experimental.pallas import tpu as pltpu
from jax.experimental.pallas import tpu_sc as plsc

N = 10000
E = 640000
EB = 128
KJ = 5
NC = 2
NS = 16
NW = NC * NS
RW = 157
ROWS_P = NW * RW
E_PAD = ROWS_P * EB
NIT = 31
TAIL = RW - NIT * KJ
D_PAD = 48
DEG_W = 16
N_T = 10016
ROWS_T = N_T // NS
BN = 400


def _sc_mesh():
    return plsc.VectorSubcoreMesh(core_axis_name="c", subcore_axis_name="s")


_SC_PARAMS = pltpu.CompilerParams(use_tc_tiling_on_sc=False)


def _fill_zeros(ref, rows, width):
    zv = jnp.zeros((16,), jnp.float32)

    def body(r, carry):
        for j in range(width // 16):
            ref[r, pl.ds(j * 16, 16)] = zv
        return carry
    lax.fori_loop(0, rows, body, 0)


def _deg_partials(d1, d2):
    @functools.partial(
        pl.kernel,
        mesh=_sc_mesh(),
        compiler_params=_SC_PARAMS,
        out_type=[jax.ShapeDtypeStruct((NC, N_T, DEG_W), jnp.float32),
                  jax.ShapeDtypeStruct((NC, N_T, DEG_W), jnp.float32)],
        scratch_types=[
            pltpu.VMEM((RW, EB), jnp.int32),
            pltpu.VMEM((EB, DEG_W), jnp.float32),
            pltpu.VMEM((ROWS_T, DEG_W), jnp.float32),
            pltpu.VMEM_SHARED((N_T, DEG_W), jnp.float32),
            pltpu.VMEM_SHARED((N_T, DEG_W), jnp.float32),
        ],
    )
    def kern(d1h, d2h, o1, o2, idxv, onesv, zv, t1, t2):
        cid = lax.axis_index("c")
        sid = lax.axis_index("s")
        wid = sid * NC + cid
        r0 = sid * ROWS_T
        _fill_zeros(zv, ROWS_T, DEG_W)
        one = jnp.ones((16,), jnp.float32)

        def fill_ones(r, carry):
            onesv[r, pl.ds(0, 16)] = one
            return carry
        lax.fori_loop(0, EB, fill_ones, 0)
        pltpu.sync_copy(zv, t1.at[pl.ds(r0, ROWS_T)])
        pltpu.sync_copy(zv, t2.at[pl.ds(r0, ROWS_T)])
        plsc.subcore_barrier()
        for dh, tbl in ((d1h, t1), (d2h, t2)):
            pltpu.sync_copy(dh.at[pl.ds(wid * RW, RW)], idxv)

            def body(it, carry, tbl=tbl):
                for j in range(KJ):
                    pltpu.sync_copy(onesv, tbl.at[idxv.at[it * KJ + j]],
                                    add=True)
                return carry
            lax.fori_loop(0, NIT, body, 0)
            for j in range(TAIL):
                pltpu.sync_copy(onesv, tbl.at[idxv.at[NIT * KJ + j]],
                                add=True)
        plsc.subcore_barrier()
        pltpu.sync_copy(t1.at[pl.ds(r0, ROWS_T)], o1.at[cid, pl.ds(r0, ROWS_T)])
        pltpu.sync_copy(t2.at[pl.ds(r0, ROWS_T)], o2.at[cid, pl.ds(r0, ROWS_T)])

    return kern(d1, d2)


def _edge_agg_one(sh_in, dh_in, xs):
    @functools.partial(
        pl.kernel,
        mesh=_sc_mesh(),
        compiler_params=_SC_PARAMS,
        out_type=jax.ShapeDtypeStruct((NC, N_T, D_PAD), jnp.float32),
        scratch_types=[
            pltpu.VMEM((RW, EB), jnp.int32),
            pltpu.VMEM((RW, EB), jnp.int32),
            pltpu.VMEM((KJ, EB, D_PAD), jnp.float32),
            pltpu.VMEM((ROWS_T, D_PAD), jnp.float32),
            pltpu.VMEM_SHARED((N_T, D_PAD), jnp.float32),
            pltpu.SemaphoreType.DMA,
        ],
    )
    def kern(sh, dh, xh, o1, sv, dv, rowsv, zv, a1, sem):
        cid = lax.axis_index("c")
        sid = lax.axis_index("s")
        wid = sid * NC + cid
        r0 = sid * ROWS_T
        _fill_zeros(zv, ROWS_T, D_PAD)
        pltpu.sync_copy(zv, a1.at[pl.ds(r0, ROWS_T)])
        pltpu.sync_copy(sh.at[pl.ds(wid * RW, RW)], sv)
        pltpu.sync_copy(dh.at[pl.ds(wid * RW, RW)], dv)
        plsc.subcore_barrier()

        def body(it, carry):
            cps = [pltpu.async_copy(xh.at[sv.at[it * KJ + j]],
                                    rowsv.at[j], sem)
                   for j in range(KJ)]
            for c in cps:
                c.wait()
            for j in range(KJ):
                pltpu.sync_copy(rowsv.at[j], a1.at[dv.at[it * KJ + j]],
                                add=True)
            return carry

        lax.fori_loop(0, NIT, body, 0)
        cps = [pltpu.async_copy(xh.at[sv.at[NIT * KJ + j]],
                                rowsv.at[j], sem)
               for j in range(TAIL)]
        for c in cps:
            c.wait()
        for j in range(TAIL):
            pltpu.sync_copy(rowsv.at[j], a1.at[dv.at[NIT * KJ + j]],
                            add=True)
        plsc.subcore_barrier()
        pltpu.sync_copy(a1.at[pl.ds(r0, ROWS_T)], o1.at[cid, pl.ds(r0, ROWS_T)])

    return kern(sh_in, dh_in, xs)


def _enc_body(x_ref, d1_ref, d2_ref, wd_ref, bd_ref, fcw_ref, fcb_ref,
              l1w_ref, l1b_ref, wih_ref, whh_ref, bih_ref, bhh_ref,
              g1w_ref, g2w_ref, xs1_ref, xs2_ref, di1_ref, di2_ref):
    xb = x_ref[...]
    y = xb[:, :, 2:3]
    xc = xb[:, :, 3:].reshape(5 * BN, 392)
    co = jnp.maximum(
        jnp.dot(xc, wd_ref[...], preferred_element_type=jnp.float32)
        + bd_ref[...], 0.0)
    f = jnp.dot(co, fcw_ref[...], preferred_element_type=jnp.float32) + fcb_ref[...]
    l = jnp.maximum(
        jnp.dot(f, l1w_ref[...], preferred_element_type=jnp.float32)
        + l1b_ref[...], 0.0)
    xcat3 = jnp.concatenate([l.reshape(BN, 5, 40), y], axis=2)
    wih = wih_ref[...]
    whh = whh_ref[...]
    bih = bih_ref[...]
    bhh = bhh_ref[...]
    h = jnp.zeros((BN, 64), jnp.float32)
    for t in range(4):
        xt = xcat3[:, t, :]
        gi = jnp.dot(xt, wih, preferred_element_type=jnp.float32) + bih
        gh = jnp.dot(h, whh, preferred_element_type=jnp.float32) + bhh
        r = jax.nn.sigmoid(gi[:, 0:64] + gh[:, 0:64])
        z = jax.nn.sigmoid(gi[:, 64:128] + gh[:, 64:128])
        n = jnp.tanh(gi[:, 128:192] + r * gh[:, 128:192])
        h = (1.0 - z) * n + z * h
    hh = jnp.concatenate([xcat3[:, 4, :40], h], axis=1)
    pad = jnp.zeros((BN, D_PAD - 40), jnp.float32)
    for d_ref, gw_ref, xs_ref, di_ref in (
            (d1_ref, g1w_ref, xs1_ref, di1_ref),
            (d2_ref, g2w_ref, xs2_ref, di2_ref)):
        deg = d_ref[0, :, 0:1] + d_ref[1, :, 0:1] + 1.0
        dinv = lax.rsqrt(deg)
        xw = jnp.dot(hh, gw_ref[...], preferred_element_type=jnp.float32)
        xs_ref[...] = jnp.concatenate([xw * dinv, pad], axis=1)
        di_ref[...] = dinv


def _encoder(x, dp1, dp2, wd, bd, fcp, fcb, l1w, l1b,
             wihT, whhT, bih, bhh, g1w, g2w):
    full = lambda shape: pl.BlockSpec(shape, lambda i: tuple(0 for _ in shape))
    return pl.pallas_call(
        _enc_body,
        grid=(N // BN,),
        in_specs=[
            pl.BlockSpec((BN, 5, 395), lambda i: (i, 0, 0)),
            pl.BlockSpec((NC, BN, DEG_W), lambda i: (0, i, 0)),
            pl.BlockSpec((NC, BN, DEG_W), lambda i: (0, i, 0)),
            full((392, 784)),
            full((1, 784)),
            full((784, 80)),
            full((1, 80)),
            full((80, 40)),
            full((1, 40)),
            full((41, 192)),
            full((64, 192)),
            full((1, 192)),
            full((1, 192)),
            full((104, 40)),
            full((104, 40)),
        ],
        out_specs=[
            pl.BlockSpec((BN, D_PAD), lambda i: (i, 0)),
            pl.BlockSpec((BN, D_PAD), lambda i: (i, 0)),
            pl.BlockSpec((BN, 1), lambda i: (i, 0)),
            pl.BlockSpec((BN, 1), lambda i: (i, 0)),
        ],
        out_shape=[
            jax.ShapeDtypeStruct((N, D_PAD), jnp.float32),
            jax.ShapeDtypeStruct((N, D_PAD), jnp.float32),
            jax.ShapeDtypeStruct((N, 1), jnp.float32),
            jax.ShapeDtypeStruct((N, 1), jnp.float32),
        ],
    )(x, dp1, dp2, wd, bd, fcp, fcb, l1w, l1b,
      wihT, whhT, bih, bhh, g1w, g2w)


def _comb_body(a1_ref, a2_ref, xs1_ref, xs2_ref, di1_ref, di2_ref,
               b1_ref, b2_ref, mw1_ref, mw2_ref, mb_ref, out_ref):
    g1 = jnp.maximum(
        (a1_ref[0] + a1_ref[1] + xs1_ref[...])[:, :40] * di1_ref[...]
        + b1_ref[...], 0.0)
    g2 = jnp.maximum(
        (a2_ref[0] + a2_ref[1] + xs2_ref[...])[:, :40] * di2_ref[...]
        + b2_ref[...], 0.0)
    out_ref[...] = (
        jnp.dot(g1, mw1_ref[...], preferred_element_type=jnp.float32)
        + jnp.dot(g2, mw2_ref[...], preferred_element_type=jnp.float32)
        + mb_ref[...])


def _combine(a1, a2, xs1, xs2, di1, di2, b1, b2, mw1, mw2, mb):
    return pl.pallas_call(
        _comb_body,
        out_shape=jax.ShapeDtypeStruct((N, 1), jnp.float32),
    )(a1, a2, xs1, xs2, di1, di2, b1, b2, mw1, mw2, mb)


def kernel(x, edge_index, feat_edge_index, conv_w, conv_b, fc_w, fc_b,
           lin1_w, lin1_b, gru_w_ih, gru_w_hh, gru_b_ih, gru_b_hh,
           gcn1_w, gcn1_b, gcn2_w, gcn2_b, mlp_w, mlp_b):
    f32 = jnp.float32
    wd = jnp.einsum("pq,ck->pkqc", jnp.eye(49, dtype=f32),
                    conv_w[:, 0, :]).reshape(392, 784)
    bd = jnp.tile(conv_b, 49)[None, :]
    fcp = fc_w.reshape(16, 49, 80).transpose(1, 0, 2).reshape(784, 80)

    pad_src = jnp.zeros((E_PAD - E,), jnp.int32)
    pad_dst = jnp.full((E_PAD - E,), N, jnp.int32)
    s1 = jnp.concatenate([edge_index[0], pad_src]).reshape(ROWS_P, EB)
    d1 = jnp.concatenate([edge_index[1], pad_dst]).reshape(ROWS_P, EB)
    s2 = jnp.concatenate([feat_edge_index[0], pad_src]).reshape(ROWS_P, EB)
    d2 = jnp.concatenate([feat_edge_index[1], pad_dst]).reshape(ROWS_P, EB)

    dp1, dp2 = _deg_partials(d1, d2)
    xs1, xs2, di1, di2 = _encoder(
        x, dp1[:, :N], dp2[:, :N], wd, bd, fcp, fc_b[None],
        lin1_w, lin1_b[None], gru_w_ih.T, gru_w_hh.T, gru_b_ih[None],
        gru_b_hh[None], gcn1_w, gcn2_w)
    a1 = _edge_agg_one(s1, d1, xs1)
    a2 = _edge_agg_one(s2, d2, xs2)
    return _combine(a1[:, :N], a2[:, :N], xs1, xs2, di1, di2,
                    gcn1_b[None], gcn2_b[None],
                    mlp_w[:40], mlp_w[40:], mlp_b[None])


# speedup vs baseline: 1.4033x; 1.0840x over previous
import functools

import jax
import jax.numpy as jnp
from jax import lax
from jax.experimental import pallas as pl
from jax.experimental.pallas import tpu as pltpu
from jax.experimental.pallas import tpu_sc as plsc

N = 10000
E = 640000
EB = 128
KJ = 5
NC = 2
NS = 16
NW = NC * NS
RW = 157
ROWS_P = NW * RW
E_PAD = ROWS_P * EB
NIT = 31
TAIL = RW - NIT * KJ
KJA = 3
NITA = 52
TAILA = RW - NITA * KJA
ZR = 64
D_PAD = 48
DEG_W = 16
N_T = 10016
ROWS_T = N_T // NS
BN = 400


def _sc_mesh():
    return plsc.VectorSubcoreMesh(core_axis_name="c", subcore_axis_name="s")


_SC_PARAMS = pltpu.CompilerParams(use_tc_tiling_on_sc=False)


def _fill_zeros(ref, rows, width):
    zv = jnp.zeros((16,), jnp.float32)

    def body(r, carry):
        for j in range(width // 16):
            ref[r, pl.ds(j * 16, 16)] = zv
        return carry
    lax.fori_loop(0, rows, body, 0)


def _deg_partials(d1, d2):
    @functools.partial(
        pl.kernel,
        mesh=_sc_mesh(),
        compiler_params=_SC_PARAMS,
        out_type=[jax.ShapeDtypeStruct((NC, N_T, DEG_W), jnp.float32),
                  jax.ShapeDtypeStruct((NC, N_T, DEG_W), jnp.float32)],
        scratch_types=[
            pltpu.VMEM((RW, EB), jnp.int32),
            pltpu.VMEM((EB, DEG_W), jnp.float32),
            pltpu.VMEM((ROWS_T, DEG_W), jnp.float32),
            pltpu.VMEM_SHARED((N_T, DEG_W), jnp.float32),
            pltpu.VMEM_SHARED((N_T, DEG_W), jnp.float32),
        ],
    )
    def kern(d1h, d2h, o1, o2, idxv, onesv, zv, t1, t2):
        cid = lax.axis_index("c")
        sid = lax.axis_index("s")
        wid = sid * NC + cid
        r0 = sid * ROWS_T
        _fill_zeros(zv, ROWS_T, DEG_W)
        one = jnp.ones((16,), jnp.float32)

        def fill_ones(r, carry):
            onesv[r, pl.ds(0, 16)] = one
            return carry
        lax.fori_loop(0, EB, fill_ones, 0)
        pltpu.sync_copy(zv, t1.at[pl.ds(r0, ROWS_T)])
        pltpu.sync_copy(zv, t2.at[pl.ds(r0, ROWS_T)])
        plsc.subcore_barrier()
        for dh, tbl in ((d1h, t1), (d2h, t2)):
            pltpu.sync_copy(dh.at[pl.ds(wid * RW, RW)], idxv)

            def body(it, carry, tbl=tbl):
                for j in range(KJ):
                    pltpu.sync_copy(onesv, tbl.at[idxv.at[it * KJ + j]],
                                    add=True)
                return carry
            lax.fori_loop(0, NIT, body, 0)
            for j in range(TAIL):
                pltpu.sync_copy(onesv, tbl.at[idxv.at[NIT * KJ + j]],
                                add=True)
        plsc.subcore_barrier()
        pltpu.sync_copy(t1.at[pl.ds(r0, ROWS_T)], o1.at[cid, pl.ds(r0, ROWS_T)])
        pltpu.sync_copy(t2.at[pl.ds(r0, ROWS_T)], o2.at[cid, pl.ds(r0, ROWS_T)])

    return kern(d1, d2)


def _edge_agg_one(sh_in, dh_in, xs):
    @functools.partial(
        pl.kernel,
        mesh=_sc_mesh(),
        compiler_params=_SC_PARAMS,
        out_type=jax.ShapeDtypeStruct((NC, N_T, D_PAD), jnp.float32),
        scratch_types=[
            pltpu.VMEM((RW, EB), jnp.int32),
            pltpu.VMEM((RW, EB), jnp.int32),
            pltpu.VMEM((2, KJA, EB, D_PAD), jnp.float32),
            pltpu.VMEM((ZR, D_PAD), jnp.float32),
            pltpu.VMEM_SHARED((N_T, D_PAD), jnp.float32),
            pltpu.SemaphoreType.DMA,
            pltpu.SemaphoreType.DMA,
        ],
    )
    def kern(sh, dh, xh, o1, sv, dv, rowsv, zv, a1, sem0, sem1):
        cid = lax.axis_index("c")
        sid = lax.axis_index("s")
        wid = sid * NC + cid
        r0 = sid * ROWS_T
        _fill_zeros(zv, ZR, D_PAD)
        nfull, rem = divmod(ROWS_T, ZR)
        for zc in range(nfull):
            pltpu.sync_copy(zv, a1.at[pl.ds(r0 + zc * ZR, ZR)])
        if rem:
            pltpu.sync_copy(zv.at[pl.ds(0, rem)],
                            a1.at[pl.ds(r0 + nfull * ZR, rem)])
        pltpu.sync_copy(sh.at[pl.ds(wid * RW, RW)], sv)
        pltpu.sync_copy(dh.at[pl.ds(wid * RW, RW)], dv)
        plsc.subcore_barrier()

        def fire(itv, b, sem_b):
            for j in range(KJA):
                pltpu.async_copy(xh.at[sv.at[itv * KJA + j]],
                                 rowsv.at[b, j], sem_b)

        def drain(b, sem_b):
            for j in range(KJA):
                pltpu.make_async_copy(xh.at[sv.at[0]],
                                      rowsv.at[b, j], sem_b).wait()

        def scat(itv, b):
            for j in range(KJA):
                pltpu.sync_copy(rowsv.at[b, j], a1.at[dv.at[itv * KJA + j]],
                                add=True)

        fire(0, 0, sem0)

        def body(k, carry):
            drain(0, sem0)
            fire(2 * k + 1, 1, sem1)
            scat(2 * k, 0)
            drain(1, sem1)
            fire(2 * k + 2, 0, sem0)
            scat(2 * k + 1, 1)
            return carry

        lax.fori_loop(0, NITA // 2 - 1, body, 0)
        drain(0, sem0)
        fire(NITA - 1, 1, sem1)
        scat(NITA - 2, 0)
        for j in range(TAILA):
            pltpu.async_copy(xh.at[sv.at[NITA * KJA + j]],
                             rowsv.at[0, j], sem0)
        drain(1, sem1)
        scat(NITA - 1, 1)
        for j in range(TAILA):
            pltpu.make_async_copy(xh.at[sv.at[0]],
                                  rowsv.at[0, j], sem0).wait()
        for j in range(TAILA):
            pltpu.sync_copy(rowsv.at[0, j], a1.at[dv.at[NITA * KJA + j]],
                            add=True)
        plsc.subcore_barrier()
        pltpu.sync_copy(a1.at[pl.ds(r0, ROWS_T)], o1.at[cid, pl.ds(r0, ROWS_T)])

    return kern(sh_in, dh_in, xs)


def _enc_body(x_ref, d1_ref, d2_ref, wd_ref, bd_ref, fcw_ref, fcb_ref,
              l1w_ref, l1b_ref, wih_ref, whh_ref, bih_ref, bhh_ref,
              g1w_ref, g2w_ref, xs1_ref, xs2_ref, di1_ref, di2_ref):
    xb = x_ref[...]
    y = xb[:, :, 2:3]
    xc = xb[:, :, 3:].reshape(5 * BN, 392)
    co = jnp.maximum(
        jnp.dot(xc, wd_ref[...], preferred_element_type=jnp.float32)
        + bd_ref[...], 0.0)
    f = jnp.dot(co, fcw_ref[...], preferred_element_type=jnp.float32) + fcb_ref[...]
    l = jnp.maximum(
        jnp.dot(f, l1w_ref[...], preferred_element_type=jnp.float32)
        + l1b_ref[...], 0.0)
    xcat3 = jnp.concatenate([l.reshape(BN, 5, 40), y], axis=2)
    wih = wih_ref[...]
    whh = whh_ref[...]
    bih = bih_ref[...]
    bhh = bhh_ref[...]
    h = jnp.zeros((BN, 64), jnp.float32)
    for t in range(4):
        xt = xcat3[:, t, :]
        gi = jnp.dot(xt, wih, preferred_element_type=jnp.float32) + bih
        gh = jnp.dot(h, whh, preferred_element_type=jnp.float32) + bhh
        r = jax.nn.sigmoid(gi[:, 0:64] + gh[:, 0:64])
        z = jax.nn.sigmoid(gi[:, 64:128] + gh[:, 64:128])
        n = jnp.tanh(gi[:, 128:192] + r * gh[:, 128:192])
        h = (1.0 - z) * n + z * h
    hh = jnp.concatenate([xcat3[:, 4, :40], h], axis=1)
    pad = jnp.zeros((BN, D_PAD - 40), jnp.float32)
    for d_ref, gw_ref, xs_ref, di_ref in (
            (d1_ref, g1w_ref, xs1_ref, di1_ref),
            (d2_ref, g2w_ref, xs2_ref, di2_ref)):
        deg = d_ref[0, :, 0:1] + d_ref[1, :, 0:1] + 1.0
        dinv = lax.rsqrt(deg)
        xw = jnp.dot(hh, gw_ref[...], preferred_element_type=jnp.float32)
        xs_ref[...] = jnp.concatenate([xw * dinv, pad], axis=1)
        di_ref[...] = dinv


def _encoder(x, dp1, dp2, wd, bd, fcp, fcb, l1w, l1b,
             wihT, whhT, bih, bhh, g1w, g2w):
    full = lambda shape: pl.BlockSpec(shape, lambda i: tuple(0 for _ in shape))
    return pl.pallas_call(
        _enc_body,
        grid=(N // BN,),
        in_specs=[
            pl.BlockSpec((BN, 5, 395), lambda i: (i, 0, 0)),
            pl.BlockSpec((NC, BN, DEG_W), lambda i: (0, i, 0)),
            pl.BlockSpec((NC, BN, DEG_W), lambda i: (0, i, 0)),
            full((392, 784)),
            full((1, 784)),
            full((784, 80)),
            full((1, 80)),
            full((80, 40)),
            full((1, 40)),
            full((41, 192)),
            full((64, 192)),
            full((1, 192)),
            full((1, 192)),
            full((104, 40)),
            full((104, 40)),
        ],
        out_specs=[
            pl.BlockSpec((BN, D_PAD), lambda i: (i, 0)),
            pl.BlockSpec((BN, D_PAD), lambda i: (i, 0)),
            pl.BlockSpec((BN, 1), lambda i: (i, 0)),
            pl.BlockSpec((BN, 1), lambda i: (i, 0)),
        ],
        out_shape=[
            jax.ShapeDtypeStruct((N, D_PAD), jnp.float32),
            jax.ShapeDtypeStruct((N, D_PAD), jnp.float32),
            jax.ShapeDtypeStruct((N, 1), jnp.float32),
            jax.ShapeDtypeStruct((N, 1), jnp.float32),
        ],
    )(x, dp1, dp2, wd, bd, fcp, fcb, l1w, l1b,
      wihT, whhT, bih, bhh, g1w, g2w)


def _comb_body(a1_ref, a2_ref, xs1_ref, xs2_ref, di1_ref, di2_ref,
               b1_ref, b2_ref, mw1_ref, mw2_ref, mb_ref, out_ref):
    g1 = jnp.maximum(
        (a1_ref[0] + a1_ref[1] + xs1_ref[...])[:, :40] * di1_ref[...]
        + b1_ref[...], 0.0)
    g2 = jnp.maximum(
        (a2_ref[0] + a2_ref[1] + xs2_ref[...])[:, :40] * di2_ref[...]
        + b2_ref[...], 0.0)
    out_ref[...] = (
        jnp.dot(g1, mw1_ref[...], preferred_element_type=jnp.float32)
        + jnp.dot(g2, mw2_ref[...], preferred_element_type=jnp.float32)
        + mb_ref[...])


def _combine(a1, a2, xs1, xs2, di1, di2, b1, b2, mw1, mw2, mb):
    return pl.pallas_call(
        _comb_body,
        out_shape=jax.ShapeDtypeStruct((N, 1), jnp.float32),
    )(a1, a2, xs1, xs2, di1, di2, b1, b2, mw1, mw2, mb)


def kernel(x, edge_index, feat_edge_index, conv_w, conv_b, fc_w, fc_b,
           lin1_w, lin1_b, gru_w_ih, gru_w_hh, gru_b_ih, gru_b_hh,
           gcn1_w, gcn1_b, gcn2_w, gcn2_b, mlp_w, mlp_b):
    f32 = jnp.float32
    wd = jnp.einsum("pq,ck->pkqc", jnp.eye(49, dtype=f32),
                    conv_w[:, 0, :]).reshape(392, 784)
    bd = jnp.tile(conv_b, 49)[None, :]
    fcp = fc_w.reshape(16, 49, 80).transpose(1, 0, 2).reshape(784, 80)

    pad_src = jnp.zeros((E_PAD - E,), jnp.int32)
    pad_dst = jnp.full((E_PAD - E,), N, jnp.int32)
    s1 = jnp.concatenate([edge_index[0], pad_src]).reshape(ROWS_P, EB)
    d1 = jnp.concatenate([edge_index[1], pad_dst]).reshape(ROWS_P, EB)
    s2 = jnp.concatenate([feat_edge_index[0], pad_src]).reshape(ROWS_P, EB)
    d2 = jnp.concatenate([feat_edge_index[1], pad_dst]).reshape(ROWS_P, EB)

    dp1, dp2 = _deg_partials(d1, d2)
    xs1, xs2, di1, di2 = _encoder(
        x, dp1[:, :N], dp2[:, :N], wd, bd, fcp, fc_b[None],
        lin1_w, lin1_b[None], gru_w_ih.T, gru_w_hh.T, gru_b_ih[None],
        gru_b_hh[None], gcn1_w, gcn2_w)
    a1 = _edge_agg_one(s1, d1, xs1)
    a2 = _edge_agg_one(s2, d2, xs2)
    return _combine(a1[:, :N], a2[:, :N], xs1, xs2, di1, di2,
                    gcn1_b[None], gcn2_b[None],
                    mlp_w[:40], mlp_w[40:], mlp_b[None])
